# baseline (speedup 1.0000x reference)
import functools

import jax
import jax.numpy as jnp
from jax import lax
from jax.experimental import pallas as pl
from jax.experimental.pallas import tpu as pltpu
from jax.experimental.pallas import tpu_sc as plsc

_N = 2048
_D = 1024
_E = 8
_MG = 128
_NBLK = (_N * 2) // _MG + _E
_PAD_N = _NBLK * _MG
_NC, _NS = 2, 16
_NW = _NC * _NS
_TPW = _N // _NW


def _router_body(x_ref, rw_ref, rb_ref, d1_ref, d2_ref, p1_ref, p2_ref,
                 be_ref):
    n, d = x_ref.shape
    e = rw_ref.shape[1]
    x = x_ref[...]
    logits = jnp.dot(x, rw_ref[...], preferred_element_type=jnp.float32)
    logits = logits + rb_ref[...]
    p = jax.nn.softmax(logits, axis=-1)
    iota = lax.broadcasted_iota(jnp.int32, (n, e), 1)
    m1 = jnp.max(p, axis=-1, keepdims=True)
    i1 = jnp.min(jnp.where(p == m1, iota, e), axis=-1, keepdims=True)
    pm = jnp.where(iota == i1, -jnp.inf, p)
    m2 = jnp.max(pm, axis=-1, keepdims=True)
    i2 = jnp.min(jnp.where(pm == m2, iota, e), axis=-1, keepdims=True)
    h1 = (iota == i1).astype(jnp.float32)
    h2 = (iota == i2).astype(jnp.float32)
    s = m1 + m2
    p1_ref[...] = m1 / s
    p2_ref[...] = m2 / s
    hs = h1 + h2
    c = 128
    rio = lax.broadcasted_iota(jnp.int32, (c, c), 0)
    cio = lax.broadcasted_iota(jnp.int32, (c, c), 1)
    lst = (rio > cio).astype(jnp.float32)
    carry = jnp.zeros((1, e), jnp.float32)
    chunks = []
    for ci in range(n // c):
        blk = hs[ci * c:(ci + 1) * c]
        intra = jnp.dot(lst, blk, preferred_element_type=jnp.float32)
        chunks.append(intra + carry)
        carry = carry + jnp.sum(blk, axis=0, keepdims=True)
    cex = jnp.concatenate(chunks, axis=0)
    counts = carry
    padded = jnp.ceil(counts / _MG) * _MG
    re8 = lax.broadcasted_iota(jnp.int32, (e, e), 0)
    ce8 = lax.broadcasted_iota(jnp.int32, (e, e), 1)
    upper = (re8 <= ce8).astype(jnp.float32)
    ends = jnp.dot(padded, upper, preferred_element_type=jnp.float32)
    offs = ends - padded
    d1_ref[...] = jnp.sum(h1 * (offs + cex), axis=-1,
                          keepdims=True).astype(jnp.int32)
    d2_ref[...] = jnp.sum(h2 * (offs + cex), axis=-1,
                          keepdims=True).astype(jnp.int32)
    mio = lax.broadcasted_iota(jnp.int32, (_NBLK, e), 0).astype(jnp.float32)
    ge = (mio * _MG >= ends).astype(jnp.int32)
    be_ref[...] = jnp.minimum(jnp.sum(ge, axis=-1, keepdims=True), e - 1)


def _gemm_body(be_ref, xs_ref, w_ref, y_ref):
    del be_ref
    y_ref[...] = lax.dot_general(
        xs_ref[...], w_ref[0], (((1,), (1,)), ((), ())),
        preferred_element_type=jnp.float32)


def _combine_body(a_ref, b_ref, p1_ref, p2_ref, o_ref):
    o_ref[...] = p1_ref[...] * a_ref[...] + p2_ref[...] * b_ref[...]


def _sc_mesh():
    return plsc.VectorSubcoreMesh(core_axis_name="c", subcore_axis_name="s",
                                  num_cores=_NC, num_subcores=_NS)


@functools.cache
def _make_sc_scatter():
    @functools.partial(
        pl.kernel,
        out_type=jax.ShapeDtypeStruct((_PAD_N, _D), jnp.float32),
        mesh=_sc_mesh(),
        scratch_types=[
            pltpu.VMEM((_TPW, _D), jnp.float32),
            pltpu.VMEM((_TPW,), jnp.int32),
            pltpu.VMEM((_TPW,), jnp.int32),
            pltpu.SemaphoreType.DMA,
        ],
    )
    def _sc_scatter(x_hbm, d1_hbm, d2_hbm, sorted_hbm, rows_v, i1_v, i2_v,
                    sem):
        wid = lax.axis_index("s") * _NC + lax.axis_index("c")
        base = wid * _TPW
        pltpu.sync_copy(x_hbm.at[pl.ds(base, _TPW)], rows_v)
        pltpu.sync_copy(d1_hbm.at[pl.ds(base, _TPW)], i1_v)
        pltpu.sync_copy(d2_hbm.at[pl.ds(base, _TPW)], i2_v)
        c1 = pltpu.async_copy(rows_v, sorted_hbm.at[i1_v], sem)
        c2 = pltpu.async_copy(rows_v, sorted_hbm.at[i2_v], sem)
        c1.wait()
        c2.wait()

    return _sc_scatter


@functools.cache
def _make_sc_gather():
    @functools.partial(
        pl.kernel,
        out_type=(jax.ShapeDtypeStruct((_N, _D), jnp.float32),
                  jax.ShapeDtypeStruct((_N, _D), jnp.float32)),
        mesh=_sc_mesh(),
        scratch_types=[
            pltpu.VMEM((_TPW, _D), jnp.float32),
            pltpu.VMEM((_TPW,), jnp.int32),
            pltpu.SemaphoreType.DMA,
        ],
    )
    def _sc_gather(y_hbm, d1_hbm, d2_hbm, a_hbm, b_hbm, rows_v, idx_v, sem):
        wid = lax.axis_index("s") * _NC + lax.axis_index("c")
        base = wid * _TPW
        pltpu.sync_copy(d1_hbm.at[pl.ds(base, _TPW)], idx_v)
        pltpu.async_copy(y_hbm.at[idx_v], rows_v, sem).wait()
        pltpu.sync_copy(rows_v, a_hbm.at[pl.ds(base, _TPW)])
        pltpu.sync_copy(d2_hbm.at[pl.ds(base, _TPW)], idx_v)
        pltpu.async_copy(y_hbm.at[idx_v], rows_v, sem).wait()
        pltpu.sync_copy(rows_v, b_hbm.at[pl.ds(base, _TPW)])

    return _sc_gather


def kernel(tokens, router_w, router_b, expert_weights):
    b, s, d = tokens.shape
    e = router_w.shape[1]
    n = b * s
    x = tokens.reshape(n, d)

    d1, d2, p1, p2, be = pl.pallas_call(
        _router_body,
        grid=(1,),
        in_specs=[
            pl.BlockSpec((n, d), lambda i: (0, 0)),
            pl.BlockSpec((d, e), lambda i: (0, 0)),
            pl.BlockSpec((1, e), lambda i: (0, 0)),
        ],
        out_specs=[
            pl.BlockSpec((n, 1), lambda i: (0, 0)),
            pl.BlockSpec((n, 1), lambda i: (0, 0)),
            pl.BlockSpec((n, 1), lambda i: (0, 0)),
            pl.BlockSpec((n, 1), lambda i: (0, 0)),
            pl.BlockSpec((_NBLK, 1), lambda i: (0, 0)),
        ],
        out_shape=[
            jax.ShapeDtypeStruct((n, 1), jnp.int32),
            jax.ShapeDtypeStruct((n, 1), jnp.int32),
            jax.ShapeDtypeStruct((n, 1), jnp.float32),
            jax.ShapeDtypeStruct((n, 1), jnp.float32),
            jax.ShapeDtypeStruct((_NBLK, 1), jnp.int32),
        ],
    )(x, router_w, router_b.reshape(1, e))

    d1f = d1.reshape(n)
    d2f = d2.reshape(n)

    sorted_x = _make_sc_scatter()(x, d1f, d2f)

    y = pl.pallas_call(
        _gemm_body,
        grid_spec=pltpu.PrefetchScalarGridSpec(
            num_scalar_prefetch=1,
            grid=(_NBLK,),
            in_specs=[
                pl.BlockSpec((_MG, d), lambda m, bep: (m, 0)),
                pl.BlockSpec((1, d, d), lambda m, bep: (bep[m], 0, 0)),
            ],
            out_specs=pl.BlockSpec((_MG, d), lambda m, bep: (m, 0)),
        ),
        out_shape=jax.ShapeDtypeStruct((_PAD_N, d), jnp.float32),
    )(be.reshape(_NBLK), sorted_x, expert_weights)

    a, bb = _make_sc_gather()(y, d1f, d2f)

    out = pl.pallas_call(
        _combine_body,
        grid=(n // 256,),
        in_specs=[
            pl.BlockSpec((256, d), lambda i: (i, 0)),
            pl.BlockSpec((256, d), lambda i: (i, 0)),
            pl.BlockSpec((256, 1), lambda i: (i, 0)),
            pl.BlockSpec((256, 1), lambda i: (i, 0)),
        ],
        out_specs=pl.BlockSpec((256, d), lambda i: (i, 0)),
        out_shape=jax.ShapeDtypeStruct((n, d), jnp.float32),
    )(a, bb, p1, p2)

    return out.reshape(b, s, d)


# speedup vs baseline: 1.0476x; 1.0476x over previous
import functools

import jax
import jax.numpy as jnp
from jax import lax
from jax.experimental import pallas as pl
from jax.experimental.pallas import tpu as pltpu
from jax.experimental.pallas import tpu_sc as plsc

_N = 2048
_D = 1024
_E = 8
_MG = 128
_NBLK = (_N * 2) // _MG + _E
_PAD_N = _NBLK * _MG
_NC, _NS = 2, 16
_NW = _NC * _NS
_TPW = _N // _NW


def _router_body(x_ref, rw_ref, rb_ref, d1_ref, d2_ref, p1_ref, p2_ref,
                 be_ref):
    n, d = x_ref.shape
    e = rw_ref.shape[1]
    x = x_ref[...]
    logits = jnp.dot(x, rw_ref[...], preferred_element_type=jnp.float32)
    logits = logits + rb_ref[...]
    p = jax.nn.softmax(logits, axis=-1)
    iota = lax.broadcasted_iota(jnp.int32, (n, e), 1)
    m1 = jnp.max(p, axis=-1, keepdims=True)
    i1 = jnp.min(jnp.where(p == m1, iota, e), axis=-1, keepdims=True)
    pm = jnp.where(iota == i1, -jnp.inf, p)
    m2 = jnp.max(pm, axis=-1, keepdims=True)
    i2 = jnp.min(jnp.where(pm == m2, iota, e), axis=-1, keepdims=True)
    h1 = (iota == i1).astype(jnp.float32)
    h2 = (iota == i2).astype(jnp.float32)
    s = m1 + m2
    p1_ref[...] = m1 / s
    p2_ref[...] = m2 / s
    hs = h1 + h2
    c = 128
    rio = lax.broadcasted_iota(jnp.int32, (c, c), 0)
    cio = lax.broadcasted_iota(jnp.int32, (c, c), 1)
    lst = (rio > cio).astype(jnp.float32)
    carry = jnp.zeros((1, e), jnp.float32)
    chunks = []
    for ci in range(n // c):
        blk = hs[ci * c:(ci + 1) * c]
        intra = jnp.dot(lst, blk, preferred_element_type=jnp.float32)
        chunks.append(intra + carry)
        carry = carry + jnp.sum(blk, axis=0, keepdims=True)
    cex = jnp.concatenate(chunks, axis=0)
    counts = carry
    padded = jnp.ceil(counts / _MG) * _MG
    re8 = lax.broadcasted_iota(jnp.int32, (e, e), 0)
    ce8 = lax.broadcasted_iota(jnp.int32, (e, e), 1)
    upper = (re8 <= ce8).astype(jnp.float32)
    ends = jnp.dot(padded, upper, preferred_element_type=jnp.float32)
    offs = ends - padded
    d1 = jnp.sum(h1 * (offs + cex), axis=-1).astype(jnp.int32)
    d2 = jnp.sum(h2 * (offs + cex), axis=-1).astype(jnp.int32)
    d1_ref[...] = d1.reshape(d1_ref.shape)
    d2_ref[...] = d2.reshape(d2_ref.shape)
    mio = lax.broadcasted_iota(jnp.int32, (_NBLK, e), 0).astype(jnp.float32)
    ge = (mio * _MG >= ends).astype(jnp.int32)
    be = jnp.minimum(jnp.sum(ge, axis=-1), e - 1)
    be_ref[...] = be.reshape(be_ref.shape)


def _gemm_body(be_ref, xs_ref, w_ref, y_ref):
    ei = be_ref[pl.program_id(0)]
    y_ref[...] = lax.dot_general(
        xs_ref[...], w_ref[ei], (((1,), (1,)), ((), ())),
        preferred_element_type=jnp.float32)


def _combine_body(a_ref, b_ref, p1_ref, p2_ref, o_ref):
    o_ref[...] = p1_ref[...] * a_ref[...] + p2_ref[...] * b_ref[...]


def _sc_mesh():
    return plsc.VectorSubcoreMesh(core_axis_name="c", subcore_axis_name="s",
                                  num_cores=_NC, num_subcores=_NS)


@functools.cache
def _make_sc_scatter():
    @functools.partial(
        pl.kernel,
        out_type=jax.ShapeDtypeStruct((_PAD_N, _D), jnp.float32),
        mesh=_sc_mesh(),
        scratch_types=[
            pltpu.VMEM((_TPW, _D), jnp.float32),
            pltpu.VMEM((_TPW,), jnp.int32),
            pltpu.VMEM((_TPW,), jnp.int32),
            pltpu.SemaphoreType.DMA,
        ],
    )
    def _sc_scatter(x_hbm, d1_hbm, d2_hbm, sorted_hbm, rows_v, i1_v, i2_v,
                    sem):
        wid = lax.axis_index("s") * _NC + lax.axis_index("c")
        base = wid * _TPW
        pltpu.sync_copy(x_hbm.at[pl.ds(base, _TPW)], rows_v)
        pltpu.sync_copy(d1_hbm.at[pl.ds(base, _TPW)], i1_v)
        pltpu.sync_copy(d2_hbm.at[pl.ds(base, _TPW)], i2_v)
        c1 = pltpu.async_copy(rows_v, sorted_hbm.at[i1_v], sem)
        c2 = pltpu.async_copy(rows_v, sorted_hbm.at[i2_v], sem)
        c1.wait()
        c2.wait()

    return _sc_scatter


@functools.cache
def _make_sc_gather():
    @functools.partial(
        pl.kernel,
        out_type=(jax.ShapeDtypeStruct((_N, _D), jnp.float32),
                  jax.ShapeDtypeStruct((_N, _D), jnp.float32)),
        mesh=_sc_mesh(),
        scratch_types=[
            pltpu.VMEM((_TPW, _D), jnp.float32),
            pltpu.VMEM((_TPW,), jnp.int32),
            pltpu.SemaphoreType.DMA,
        ],
    )
    def _sc_gather(y_hbm, d1_hbm, d2_hbm, a_hbm, b_hbm, rows_v, idx_v, sem):
        wid = lax.axis_index("s") * _NC + lax.axis_index("c")
        base = wid * _TPW
        pltpu.sync_copy(d1_hbm.at[pl.ds(base, _TPW)], idx_v)
        pltpu.async_copy(y_hbm.at[idx_v], rows_v, sem).wait()
        pltpu.sync_copy(rows_v, a_hbm.at[pl.ds(base, _TPW)])
        pltpu.sync_copy(d2_hbm.at[pl.ds(base, _TPW)], idx_v)
        pltpu.async_copy(y_hbm.at[idx_v], rows_v, sem).wait()
        pltpu.sync_copy(rows_v, b_hbm.at[pl.ds(base, _TPW)])

    return _sc_gather


def kernel(tokens, router_w, router_b, expert_weights):
    b, s, d = tokens.shape
    e = router_w.shape[1]
    n = b * s
    x = tokens.reshape(n, d)

    d1, d2, p1, p2, be = pl.pallas_call(
        _router_body,
        grid=(1,),
        in_specs=[
            pl.BlockSpec((n, d), lambda i: (0, 0)),
            pl.BlockSpec((d, e), lambda i: (0, 0)),
            pl.BlockSpec((1, e), lambda i: (0, 0)),
        ],
        out_specs=[
            pl.BlockSpec((n // 128, 128), lambda i: (0, 0)),
            pl.BlockSpec((n // 128, 128), lambda i: (0, 0)),
            pl.BlockSpec((n, 1), lambda i: (0, 0)),
            pl.BlockSpec((n, 1), lambda i: (0, 0)),
            pl.BlockSpec((1, _NBLK), lambda i: (0, 0)),
        ],
        out_shape=[
            jax.ShapeDtypeStruct((n // 128, 128), jnp.int32),
            jax.ShapeDtypeStruct((n // 128, 128), jnp.int32),
            jax.ShapeDtypeStruct((n, 1), jnp.float32),
            jax.ShapeDtypeStruct((n, 1), jnp.float32),
            jax.ShapeDtypeStruct((1, _NBLK), jnp.int32),
        ],
    )(x, router_w, router_b.reshape(1, e))

    d1f = d1.reshape(n)
    d2f = d2.reshape(n)

    sorted_x = _make_sc_scatter()(x, d1f, d2f)

    y = pl.pallas_call(
        _gemm_body,
        grid_spec=pltpu.PrefetchScalarGridSpec(
            num_scalar_prefetch=1,
            grid=(_NBLK,),
            in_specs=[
                pl.BlockSpec((_MG, d), lambda m, bep: (m, 0)),
                pl.BlockSpec((_E, d, d), lambda m, bep: (0, 0, 0)),
            ],
            out_specs=pl.BlockSpec((_MG, d), lambda m, bep: (m, 0)),
        ),
        out_shape=jax.ShapeDtypeStruct((_PAD_N, d), jnp.float32),
    )(be.reshape(_NBLK), sorted_x, expert_weights)

    a, bb = _make_sc_gather()(y, d1f, d2f)

    out = pl.pallas_call(
        _combine_body,
        grid=(n // 256,),
        in_specs=[
            pl.BlockSpec((256, d), lambda i: (i, 0)),
            pl.BlockSpec((256, d), lambda i: (i, 0)),
            pl.BlockSpec((256, 1), lambda i: (i, 0)),
            pl.BlockSpec((256, 1), lambda i: (i, 0)),
        ],
        out_specs=pl.BlockSpec((256, d), lambda i: (i, 0)),
        out_shape=jax.ShapeDtypeStruct((n, d), jnp.float32),
    )(a, bb, p1, p2)

    return out.reshape(b, s, d)


# speedup vs baseline: 2.3730x; 2.2651x over previous
import jax
import jax.numpy as jnp
from jax.experimental import pallas as pl


def _moe_body(x_ref, rw_ref, rb_ref, w_ref, o_ref):
    m, d = x_ref.shape
    e = rw_ref.shape[1]
    xb = x_ref[...]
    logits = jnp.dot(xb, rw_ref[...], preferred_element_type=jnp.float32)
    logits = logits + rb_ref[...]
    p = jax.nn.softmax(logits, axis=-1)
    iota = jax.lax.broadcasted_iota(jnp.int32, (m, e), 1)
    m1 = jnp.max(p, axis=-1, keepdims=True)
    i1 = jnp.min(jnp.where(p == m1, iota, e), axis=-1, keepdims=True)
    pm = jnp.where(iota == i1, -jnp.inf, p)
    m2 = jnp.max(pm, axis=-1, keepdims=True)
    i2 = jnp.min(jnp.where(pm == m2, iota, e), axis=-1, keepdims=True)
    comb = jnp.where((iota == i1) | (iota == i2), p, 0.0) / (m1 + m2)
    acc = jnp.zeros((m, d), jnp.float32)
    for ei in range(e):
        y = jax.lax.dot_general(
            xb, w_ref[ei], (((1,), (1,)), ((), ())),
            preferred_element_type=jnp.float32)
        acc = acc + comb[:, ei:ei + 1] * y
    o_ref[...] = acc


def kernel(tokens, router_w, router_b, expert_weights):
    b, s, d = tokens.shape
    e = router_w.shape[1]
    x = tokens.reshape(b * s, d)
    n = b * s
    M = 1024
    out = pl.pallas_call(
        _moe_body,
        grid=(n // M,),
        in_specs=[
            pl.BlockSpec((M, d), lambda i: (i, 0)),
            pl.BlockSpec((d, e), lambda i: (0, 0)),
            pl.BlockSpec((1, e), lambda i: (0, 0)),
            pl.BlockSpec((e, d, d), lambda i: (0, 0, 0)),
        ],
        out_specs=pl.BlockSpec((M, d), lambda i: (i, 0)),
        out_shape=jax.ShapeDtypeStruct((n, d), jnp.float32),
    )(x, router_w, router_b.reshape(1, e), expert_weights)
    return out.reshape(b, s, d)
